# baseline (speedup 1.0000x reference)
import functools

import jax
import jax.numpy as jnp
from jax import lax
from jax.experimental import pallas as pl
from jax.experimental.pallas import tpu as pltpu
from jax.experimental.pallas import tpu_sc as plsc

_B = 16384
_D = 32
_PACK = 4
_WIDE = _D * _PACK
_NC = 2
_NS = 16
_NW = _NC * _NS
_BPW = _B // _NW
_CHUNK = 256
_NCHUNK = _BPW // _CHUNK


def _sc_gather2(uemb_w, iemb_w, uidx_c, iidx_c):
    mesh = plsc.VectorSubcoreMesh(core_axis_name="c", subcore_axis_name="s")

    @functools.partial(
        pl.kernel,
        mesh=mesh,
        out_type=(
            jax.ShapeDtypeStruct((_B, _WIDE), jnp.float32),
            jax.ShapeDtypeStruct((_B, _WIDE), jnp.float32),
        ),
        scratch_types=[
            pltpu.VMEM((_CHUNK,), jnp.int32),
            pltpu.VMEM((_CHUNK, _WIDE), jnp.float32),
            pltpu.VMEM((_CHUNK,), jnp.int32),
            pltpu.VMEM((_CHUNK, _WIDE), jnp.float32),
            pltpu.SemaphoreType.DMA,
            pltpu.SemaphoreType.DMA,
        ],
    )
    def k(uemb_hbm, iemb_hbm, uidx_hbm, iidx_hbm, ue_out, ie_out,
          uidx_v, urows_v, iidx_v, irows_v, sem_u, sem_i):
        wid = lax.axis_index("s") * _NC + lax.axis_index("c")
        base = wid * _BPW
        for c in range(_NCHUNK):
            cb = base + c * _CHUNK
            pltpu.sync_copy(uidx_hbm.at[pl.ds(cb, _CHUNK)], uidx_v)
            pltpu.sync_copy(iidx_hbm.at[pl.ds(cb, _CHUNK)], iidx_v)
            cu = pltpu.async_copy(uemb_hbm.at[uidx_v], urows_v, sem_u)
            ci = pltpu.async_copy(iemb_hbm.at[iidx_v], irows_v, sem_i)
            cu.wait()
            ci.wait()
            pltpu.sync_copy(urows_v, ue_out.at[pl.ds(cb, _CHUNK)])
            pltpu.sync_copy(irows_v, ie_out.at[pl.ds(cb, _CHUNK)])

    return k(uemb_w, iemb_w, uidx_c, iidx_c)


def _select_block(wide, sel):
    acc = jnp.zeros((wide.shape[0], _D), jnp.float32)
    for kk in range(_PACK):
        blk = wide[:, kk * _D:(kk + 1) * _D]
        acc = acc + jnp.where(sel == kk, blk, 0.0)
    return acc


def _mlp_body(uw_ref, iw_ref, usel_ref, isel_ref, w1u_ref, w1i_ref, b1_ref,
              w2_ref, b2_ref, w3_ref, b3_ref, wo_ref, bo_ref, out_ref):
    ue = _select_block(uw_ref[...], usel_ref[...])
    ie = _select_block(iw_ref[...], isel_ref[...])
    x = (jnp.dot(ue, w1u_ref[...], preferred_element_type=jnp.float32)
         + jnp.dot(ie, w1i_ref[...], preferred_element_type=jnp.float32)
         + b1_ref[...])
    x = jnp.maximum(x, 0.0)
    x = jnp.dot(x, w2_ref[...], preferred_element_type=jnp.float32) + b2_ref[...]
    x = jnp.maximum(x, 0.0)
    x = jnp.dot(x, w3_ref[...], preferred_element_type=jnp.float32) + b3_ref[...]
    x = jnp.maximum(x, 0.0)
    y = jnp.dot(x, wo_ref[...], preferred_element_type=jnp.float32) + bo_ref[...]
    out_ref[...] = y


_BLK = 2048


def _tc_mlp(uw, iw, usel, isel, W1, b1, W2, b2, W3, b3, Wout, bout):
    w1u = W1[:_D]
    w1i = W1[_D:]
    blk = lambda shape: pl.BlockSpec(shape, lambda i: (i, 0))
    full = lambda shape: pl.BlockSpec(shape, lambda i: (0, 0))
    out = pl.pallas_call(
        _mlp_body,
        grid=(_B // _BLK,),
        in_specs=[
            blk((_BLK, _WIDE)), blk((_BLK, _WIDE)),
            blk((_BLK, 1)), blk((_BLK, 1)),
            full((_D, 32)), full((_D, 32)), full((1, 32)),
            full((32, 16)), full((1, 16)),
            full((16, 8)), full((1, 8)),
            full((8, 1)), full((1, 1)),
        ],
        out_specs=blk((_BLK, 1)),
        out_shape=jax.ShapeDtypeStruct((_B, 1), jnp.float32),
    )(uw, iw, usel[:, None], isel[:, None], w1u, w1i, b1[None, :],
      W2, b2[None, :], W3, b3[None, :], Wout, bout[None, :])
    return out[:, 0]


def kernel(user_idx, item_idx, user_emb, item_emb,
           W1, b1, W2, b2, W3, b3, Wout, bout):
    uemb_w = user_emb.reshape(-1, _WIDE)
    iemb_w = item_emb.reshape(-1, _WIDE)
    uidx_c = lax.shift_right_logical(user_idx, 2)
    iidx_c = lax.shift_right_logical(item_idx, 2)
    usel = jnp.bitwise_and(user_idx, 3)
    isel = jnp.bitwise_and(item_idx, 3)
    uw, iw = _sc_gather2(uemb_w, iemb_w, uidx_c, iidx_c)
    return _tc_mlp(uw, iw, usel, isel, W1, b1, W2, b2, W3, b3, Wout, bout)
